# baseline (speedup 1.0000x reference)
import functools

import jax
import jax.numpy as jnp
from jax import lax
from jax.experimental import pallas as pl
from jax.experimental.pallas import tpu as pltpu
from jax.experimental.pallas import tpu_sc as plsc

THETA_F = 1.15
FILTER_F = 1e-08
NEG_BIG = -1e30

NC = 2
NS = 16
NW = NC * NS
LANES = 16
CHUNK = 128


def _tile_flatten(arr):
    B, C, H, W = arr.shape
    G = B * C * H // 8
    a3 = arr.reshape(G, 8, W)

    def body(ref, oref, sem):
        copies = []
        for ct in range(W // 128):
            for sl in range(8):
                c = pltpu.make_async_copy(
                    ref.at[pl.ds(0, G), pl.ds(sl, 1), pl.ds(ct * 128, 128)],
                    oref.at[pl.ds(0, G), pl.ds(ct * 8 + sl, 1), pl.ds(0, 128)],
                    sem,
                )
                c.start()
                copies.append(c)
        for c in copies:
            c.wait()

    return pl.pallas_call(
        body,
        in_specs=[pl.BlockSpec(memory_space=pl.ANY)],
        out_specs=pl.BlockSpec(memory_space=pl.ANY),
        out_shape=jax.ShapeDtypeStruct((G, W // 128 * 8, 128), jnp.float32),
        scratch_shapes=[pltpu.SemaphoreType.DMA],
    )(a3)


def _sc_depth_targets(n_per_w):
    mesh = plsc.VectorSubcoreMesh(core_axis_name="c", subcore_axis_name="s")
    n_chunks = n_per_w // CHUNK
    n = NW * n_per_w

    @functools.partial(
        pl.kernel,
        mesh=mesh,
        out_type=[
            jax.ShapeDtypeStruct((n,), jnp.float32),
            jax.ShapeDtypeStruct((n,), jnp.int32),
            jax.ShapeDtypeStruct((n,), jnp.int32),
        ],
        scratch_types=[
            pltpu.VMEM((n_per_w,), jnp.int32),
            pltpu.VMEM((n_per_w,), jnp.int32),
            pltpu.VMEM((n_per_w,), jnp.float32),
            pltpu.VMEM((n_per_w,), jnp.float32),
            pltpu.VMEM((n_per_w,), jnp.float32),
            pltpu.SemaphoreType.DMA,
        ],
    )
    def k1(d_hbm, ia_hbm, ib_hbm, t_hbm, qa_hbm, qb_hbm,
           ia_v, ib_v, za_v, zb_v, t_v, sem):
        wid = lax.axis_index("s") * NC + lax.axis_index("c")
        base = wid * n_per_w
        pltpu.sync_copy(ia_hbm.at[pl.ds(base, n_per_w)], ia_v)
        pltpu.sync_copy(ib_hbm.at[pl.ds(base, n_per_w)], ib_v)
        handles = []
        for j in range(n_chunks):
            for iv in (ia_v, ib_v):
                for k in range(CHUNK // LANES):
                    sl = pl.ds(j * CHUNK + k * LANES, LANES)
                    f = iv[sl]
                    q = ((f & jnp.int32(-4096))
                         | (((f >> 7) & jnp.int32(3)) << 10)
                         | (((f >> 9) & jnp.int32(7)) << 7)
                         | (f & jnp.int32(127)))
                    iv[sl] = q
            s = pl.ds(j * CHUNK, CHUNK)
            handles.append(pltpu.async_copy(d_hbm.at[ia_v.at[s]], za_v.at[s], sem))
            handles.append(pltpu.async_copy(d_hbm.at[ib_v.at[s]], zb_v.at[s], sem))
        one = jnp.float32(1.0)
        neg_one = jnp.float32(-1.0)
        zero = jnp.float32(0.0)
        for j in range(n_chunks):
            handles[2 * j].wait()
            handles[2 * j + 1].wait()
            for k in range(CHUNK // LANES):
                s = pl.ds(j * CHUNK + k * LANES, LANES)
                za = za_v[s]
                zb = zb_v[s]
                keep = (za > FILTER_F) | (zb > FILTER_F)
                t = jnp.where(za / zb > THETA_F, neg_one,
                              jnp.where(zb / za > THETA_F, one, zero))
                t_v[s] = jnp.where(keep, t, zero)
        pltpu.sync_copy(t_v, t_hbm.at[pl.ds(base, n_per_w)])
        pltpu.sync_copy(ia_v, qa_hbm.at[pl.ds(base, n_per_w)])
        pltpu.sync_copy(ib_v, qb_hbm.at[pl.ds(base, n_per_w)])

    return k1


def _sc_pred_logits(n_per_w):
    mesh = plsc.VectorSubcoreMesh(core_axis_name="c", subcore_axis_name="s")
    n_chunks = n_per_w // CHUNK
    n = NW * n_per_w

    @functools.partial(
        pl.kernel,
        mesh=mesh,
        out_type=jax.ShapeDtypeStruct((n,), jnp.float32),
        scratch_types=[
            pltpu.VMEM((n_per_w,), jnp.int32),
            pltpu.VMEM((n_per_w,), jnp.int32),
            pltpu.VMEM((n_per_w,), jnp.float32),
            pltpu.VMEM((n_per_w,), jnp.float32),
            pltpu.VMEM((n_per_w,), jnp.float32),
            pltpu.VMEM((n_per_w,), jnp.float32),
            pltpu.SemaphoreType.DMA,
        ],
    )
    def k2(p_hbm, qa_hbm, qb_hbm, t_hbm, x_hbm,
           qa_v, qb_v, pa_v, pb_v, t_v, x_v, sem):
        wid = lax.axis_index("s") * NC + lax.axis_index("c")
        base = wid * n_per_w
        pltpu.sync_copy(qa_hbm.at[pl.ds(base, n_per_w)], qa_v)
        pltpu.sync_copy(qb_hbm.at[pl.ds(base, n_per_w)], qb_v)
        handles = []
        for j in range(n_chunks):
            s = pl.ds(j * CHUNK, CHUNK)
            handles.append(pltpu.async_copy(p_hbm.at[qa_v.at[s]], pa_v.at[s], sem))
            handles.append(pltpu.async_copy(p_hbm.at[qb_v.at[s]], pb_v.at[s], sem))
        pltpu.sync_copy(t_hbm.at[pl.ds(base, n_per_w)], t_v)
        zero = jnp.float32(0.0)
        for j in range(n_chunks):
            handles[2 * j].wait()
            handles[2 * j + 1].wait()
            for k in range(CHUNK // LANES):
                s = pl.ds(j * CHUNK + k * LANES, LANES)
                t = t_v[s]
                x = jnp.where(t != zero, -t * (pa_v[s] - pb_v[s]),
                              jnp.float32(NEG_BIG))
                x_v[s] = x
        pltpu.sync_copy(x_v, x_hbm.at[pl.ds(base, n_per_w)])

    return k2


def _softplus_sum(x_ref, o_ref):
    x = x_ref[...]
    o_ref[...] = jnp.sum(jnp.log1p(jnp.exp(x))).reshape(1, 1)


def kernel(depth, pred, idx_A, idx_B):
    n = idx_A.shape[0]
    n_per_w = n // NW
    d_flat = _tile_flatten(depth).reshape(-1)
    t, qa, qb = _sc_depth_targets(n_per_w)(d_flat, idx_A, idx_B)
    p_flat = _tile_flatten(pred).reshape(-1)
    x = _sc_pred_logits(n_per_w)(p_flat, qa, qb, t)
    loss = pl.pallas_call(
        _softplus_sum,
        out_shape=jax.ShapeDtypeStruct((1, 1), jnp.float32),
    )(x.reshape(n // CHUNK, CHUNK))
    return loss[0, 0]


# speedup vs baseline: 13.9722x; 13.9722x over previous
import functools

import jax
import jax.numpy as jnp
from jax import lax
from jax.experimental import pallas as pl
from jax.experimental.pallas import tpu as pltpu
from jax.experimental.pallas import tpu_sc as plsc

THETA_F = 1.15
FILTER_F = 1e-08
NEG_BIG = -1e30

NC = 2
NS = 16
NW = NC * NS
LANES = 16
CHUNK = 128


def _tile_flatten(arr):
    B, C, H, W = arr.shape
    G = C * H // 8
    a4 = arr.reshape(B, G, 8, W)

    def body(ref, oref, sem):
        copies = []
        for ct in range(W // 128):
            for sl in range(8):
                c = pltpu.make_async_copy(
                    ref.at[0, pl.ds(0, G), pl.ds(sl, 1), pl.ds(ct * 128, 128)],
                    oref.at[0, pl.ds(0, G), pl.ds(ct * 8 + sl, 1), pl.ds(0, 128)],
                    sem,
                )
                c.start()
                copies.append(c)
        for c in copies:
            c.wait()

    return pl.pallas_call(
        body,
        grid=(B,),
        in_specs=[pl.BlockSpec((1, G, 8, W), lambda i: (i, 0, 0, 0))],
        out_specs=pl.BlockSpec((1, G, W // 128 * 8, 128),
                               lambda i: (i, 0, 0, 0)),
        out_shape=jax.ShapeDtypeStruct((B, G, W // 128 * 8, 128),
                                       jnp.float32),
        scratch_shapes=[pltpu.SemaphoreType.DMA],
    )(a4)


def _sc_depth_targets(n_per_w):
    mesh = plsc.VectorSubcoreMesh(core_axis_name="c", subcore_axis_name="s")
    n_chunks = n_per_w // CHUNK
    n = NW * n_per_w

    @functools.partial(
        pl.kernel,
        mesh=mesh,
        out_type=[
            jax.ShapeDtypeStruct((n,), jnp.float32),
            jax.ShapeDtypeStruct((n,), jnp.int32),
            jax.ShapeDtypeStruct((n,), jnp.int32),
        ],
        scratch_types=[
            pltpu.VMEM((n_per_w,), jnp.int32),
            pltpu.VMEM((n_per_w,), jnp.int32),
            pltpu.VMEM((n_per_w,), jnp.float32),
            pltpu.VMEM((n_per_w,), jnp.float32),
            pltpu.VMEM((n_per_w,), jnp.float32),
            pltpu.SemaphoreType.DMA,
        ],
    )
    def k1(d_hbm, ia_hbm, ib_hbm, t_hbm, qa_hbm, qb_hbm,
           ia_v, ib_v, za_v, zb_v, t_v, sem):
        wid = lax.axis_index("s") * NC + lax.axis_index("c")
        base = wid * n_per_w
        pltpu.sync_copy(ia_hbm.at[pl.ds(base, n_per_w)], ia_v)
        pltpu.sync_copy(ib_hbm.at[pl.ds(base, n_per_w)], ib_v)
        handles = []
        for j in range(n_chunks):
            for iv in (ia_v, ib_v):
                for k in range(CHUNK // LANES):
                    sl = pl.ds(j * CHUNK + k * LANES, LANES)
                    f = iv[sl]
                    q = ((f & jnp.int32(-4096))
                         | (((f >> 7) & jnp.int32(3)) << 10)
                         | (((f >> 9) & jnp.int32(7)) << 7)
                         | (f & jnp.int32(127)))
                    iv[sl] = q
            s = pl.ds(j * CHUNK, CHUNK)
            handles.append(pltpu.async_copy(d_hbm.at[ia_v.at[s]], za_v.at[s], sem))
            handles.append(pltpu.async_copy(d_hbm.at[ib_v.at[s]], zb_v.at[s], sem))
        one = jnp.float32(1.0)
        neg_one = jnp.float32(-1.0)
        zero = jnp.float32(0.0)
        for j in range(n_chunks):
            handles[2 * j].wait()
            handles[2 * j + 1].wait()
            for k in range(CHUNK // LANES):
                s = pl.ds(j * CHUNK + k * LANES, LANES)
                za = za_v[s]
                zb = zb_v[s]
                keep = (za > FILTER_F) | (zb > FILTER_F)
                t = jnp.where(za / zb > THETA_F, neg_one,
                              jnp.where(zb / za > THETA_F, one, zero))
                t_v[s] = jnp.where(keep, t, zero)
        pltpu.sync_copy(t_v, t_hbm.at[pl.ds(base, n_per_w)])
        pltpu.sync_copy(ia_v, qa_hbm.at[pl.ds(base, n_per_w)])
        pltpu.sync_copy(ib_v, qb_hbm.at[pl.ds(base, n_per_w)])

    return k1


def _sc_pred_logits(n_per_w):
    mesh = plsc.VectorSubcoreMesh(core_axis_name="c", subcore_axis_name="s")
    n_chunks = n_per_w // CHUNK
    n = NW * n_per_w

    @functools.partial(
        pl.kernel,
        mesh=mesh,
        out_type=jax.ShapeDtypeStruct((n,), jnp.float32),
        scratch_types=[
            pltpu.VMEM((n_per_w,), jnp.int32),
            pltpu.VMEM((n_per_w,), jnp.int32),
            pltpu.VMEM((n_per_w,), jnp.float32),
            pltpu.VMEM((n_per_w,), jnp.float32),
            pltpu.VMEM((n_per_w,), jnp.float32),
            pltpu.VMEM((n_per_w,), jnp.float32),
            pltpu.SemaphoreType.DMA,
        ],
    )
    def k2(p_hbm, qa_hbm, qb_hbm, t_hbm, x_hbm,
           qa_v, qb_v, pa_v, pb_v, t_v, x_v, sem):
        wid = lax.axis_index("s") * NC + lax.axis_index("c")
        base = wid * n_per_w
        pltpu.sync_copy(qa_hbm.at[pl.ds(base, n_per_w)], qa_v)
        pltpu.sync_copy(qb_hbm.at[pl.ds(base, n_per_w)], qb_v)
        handles = []
        for j in range(n_chunks):
            s = pl.ds(j * CHUNK, CHUNK)
            handles.append(pltpu.async_copy(p_hbm.at[qa_v.at[s]], pa_v.at[s], sem))
            handles.append(pltpu.async_copy(p_hbm.at[qb_v.at[s]], pb_v.at[s], sem))
        pltpu.sync_copy(t_hbm.at[pl.ds(base, n_per_w)], t_v)
        zero = jnp.float32(0.0)
        for j in range(n_chunks):
            handles[2 * j].wait()
            handles[2 * j + 1].wait()
            for k in range(CHUNK // LANES):
                s = pl.ds(j * CHUNK + k * LANES, LANES)
                t = t_v[s]
                x = jnp.where(t != zero, -t * (pa_v[s] - pb_v[s]),
                              jnp.float32(NEG_BIG))
                x_v[s] = x
        pltpu.sync_copy(x_v, x_hbm.at[pl.ds(base, n_per_w)])

    return k2


def _softplus_sum(x_ref, o_ref):
    x = x_ref[...]
    o_ref[...] = jnp.sum(jnp.log1p(jnp.exp(x))).reshape(1, 1)


def kernel(depth, pred, idx_A, idx_B):
    n = idx_A.shape[0]
    n_per_w = n // NW
    d_flat = _tile_flatten(depth).reshape(-1)
    t, qa, qb = _sc_depth_targets(n_per_w)(d_flat, idx_A, idx_B)
    p_flat = _tile_flatten(pred).reshape(-1)
    x = _sc_pred_logits(n_per_w)(p_flat, qa, qb, t)
    loss = pl.pallas_call(
        _softplus_sum,
        out_shape=jax.ShapeDtypeStruct((1, 1), jnp.float32),
    )(x.reshape(n // CHUNK, CHUNK))
    return loss[0, 0]


# speedup vs baseline: 16.5461x; 1.1842x over previous
import functools

import jax
import jax.numpy as jnp
from jax import lax
from jax.experimental import pallas as pl
from jax.experimental.pallas import tpu as pltpu
from jax.experimental.pallas import tpu_sc as plsc

THETA_F = 1.15
FILTER_F = 1e-08
NEG_BIG = -1e30

NC = 2
NS = 16
NW = NC * NS
LANES = 16
CHUNK = 128


def _tile_flatten(arr):
    B, C, H, W = arr.shape
    G = C * H // 8
    a4 = arr.reshape(B, G, 8, W)

    CT = W // 128

    def body(ref, oref):
        x = ref[0]
        x = x.reshape(G, 8, CT, 128)
        oref[0] = jnp.transpose(x, (0, 2, 1, 3)).reshape(G, CT * 8, 128)

    return pl.pallas_call(
        body,
        grid=(B,),
        in_specs=[pl.BlockSpec((1, G, 8, W), lambda i: (i, 0, 0, 0))],
        out_specs=pl.BlockSpec((1, G, CT * 8, 128),
                               lambda i: (i, 0, 0, 0)),
        out_shape=jax.ShapeDtypeStruct((B, G, CT * 8, 128),
                                       jnp.float32),
    )(a4)


def _sc_depth_targets(n_per_w):
    mesh = plsc.VectorSubcoreMesh(core_axis_name="c", subcore_axis_name="s")
    n_chunks = n_per_w // CHUNK
    n = NW * n_per_w

    @functools.partial(
        pl.kernel,
        mesh=mesh,
        out_type=[
            jax.ShapeDtypeStruct((n,), jnp.float32),
            jax.ShapeDtypeStruct((n,), jnp.int32),
            jax.ShapeDtypeStruct((n,), jnp.int32),
        ],
        scratch_types=[
            pltpu.VMEM((n_per_w,), jnp.int32),
            pltpu.VMEM((n_per_w,), jnp.int32),
            pltpu.VMEM((n_per_w,), jnp.float32),
            pltpu.VMEM((n_per_w,), jnp.float32),
            pltpu.VMEM((n_per_w,), jnp.float32),
            pltpu.SemaphoreType.DMA,
        ],
    )
    def k1(d_hbm, ia_hbm, ib_hbm, t_hbm, qa_hbm, qb_hbm,
           ia_v, ib_v, za_v, zb_v, t_v, sem):
        wid = lax.axis_index("s") * NC + lax.axis_index("c")
        base = wid * n_per_w
        pltpu.sync_copy(ia_hbm.at[pl.ds(base, n_per_w)], ia_v)
        pltpu.sync_copy(ib_hbm.at[pl.ds(base, n_per_w)], ib_v)
        handles = []
        for j in range(n_chunks):
            for iv in (ia_v, ib_v):
                for k in range(CHUNK // LANES):
                    sl = pl.ds(j * CHUNK + k * LANES, LANES)
                    f = iv[sl]
                    q = ((f & jnp.int32(-4096))
                         | (((f >> 7) & jnp.int32(3)) << 10)
                         | (((f >> 9) & jnp.int32(7)) << 7)
                         | (f & jnp.int32(127)))
                    iv[sl] = q
            s = pl.ds(j * CHUNK, CHUNK)
            handles.append(pltpu.async_copy(d_hbm.at[ia_v.at[s]], za_v.at[s], sem))
            handles.append(pltpu.async_copy(d_hbm.at[ib_v.at[s]], zb_v.at[s], sem))
        one = jnp.float32(1.0)
        neg_one = jnp.float32(-1.0)
        zero = jnp.float32(0.0)
        for j in range(n_chunks):
            handles[2 * j].wait()
            handles[2 * j + 1].wait()
            for k in range(CHUNK // LANES):
                s = pl.ds(j * CHUNK + k * LANES, LANES)
                za = za_v[s]
                zb = zb_v[s]
                keep = (za > FILTER_F) | (zb > FILTER_F)
                t = jnp.where(za / zb > THETA_F, neg_one,
                              jnp.where(zb / za > THETA_F, one, zero))
                t_v[s] = jnp.where(keep, t, zero)
        pltpu.sync_copy(t_v, t_hbm.at[pl.ds(base, n_per_w)])
        pltpu.sync_copy(ia_v, qa_hbm.at[pl.ds(base, n_per_w)])
        pltpu.sync_copy(ib_v, qb_hbm.at[pl.ds(base, n_per_w)])

    return k1


def _sc_pred_logits(n_per_w):
    mesh = plsc.VectorSubcoreMesh(core_axis_name="c", subcore_axis_name="s")
    n_chunks = n_per_w // CHUNK
    n = NW * n_per_w

    @functools.partial(
        pl.kernel,
        mesh=mesh,
        out_type=jax.ShapeDtypeStruct((n,), jnp.float32),
        scratch_types=[
            pltpu.VMEM((n_per_w,), jnp.int32),
            pltpu.VMEM((n_per_w,), jnp.int32),
            pltpu.VMEM((n_per_w,), jnp.float32),
            pltpu.VMEM((n_per_w,), jnp.float32),
            pltpu.VMEM((n_per_w,), jnp.float32),
            pltpu.VMEM((n_per_w,), jnp.float32),
            pltpu.SemaphoreType.DMA,
        ],
    )
    def k2(p_hbm, qa_hbm, qb_hbm, t_hbm, x_hbm,
           qa_v, qb_v, pa_v, pb_v, t_v, x_v, sem):
        wid = lax.axis_index("s") * NC + lax.axis_index("c")
        base = wid * n_per_w
        pltpu.sync_copy(qa_hbm.at[pl.ds(base, n_per_w)], qa_v)
        pltpu.sync_copy(qb_hbm.at[pl.ds(base, n_per_w)], qb_v)
        handles = []
        for j in range(n_chunks):
            s = pl.ds(j * CHUNK, CHUNK)
            handles.append(pltpu.async_copy(p_hbm.at[qa_v.at[s]], pa_v.at[s], sem))
            handles.append(pltpu.async_copy(p_hbm.at[qb_v.at[s]], pb_v.at[s], sem))
        pltpu.sync_copy(t_hbm.at[pl.ds(base, n_per_w)], t_v)
        zero = jnp.float32(0.0)
        for j in range(n_chunks):
            handles[2 * j].wait()
            handles[2 * j + 1].wait()
            for k in range(CHUNK // LANES):
                s = pl.ds(j * CHUNK + k * LANES, LANES)
                t = t_v[s]
                x = jnp.where(t != zero, -t * (pa_v[s] - pb_v[s]),
                              jnp.float32(NEG_BIG))
                x_v[s] = x
        pltpu.sync_copy(x_v, x_hbm.at[pl.ds(base, n_per_w)])

    return k2


def _softplus_sum(x_ref, o_ref):
    x = x_ref[...]
    o_ref[...] = jnp.sum(jnp.log1p(jnp.exp(x))).reshape(1, 1)


def kernel(depth, pred, idx_A, idx_B):
    n = idx_A.shape[0]
    n_per_w = n // NW
    d_flat = _tile_flatten(depth).reshape(-1)
    t, qa, qb = _sc_depth_targets(n_per_w)(d_flat, idx_A, idx_B)
    p_flat = _tile_flatten(pred).reshape(-1)
    x = _sc_pred_logits(n_per_w)(p_flat, qa, qb, t)
    loss = pl.pallas_call(
        _softplus_sum,
        out_shape=jax.ShapeDtypeStruct((1, 1), jnp.float32),
    )(x.reshape(n // CHUNK, CHUNK))
    return loss[0, 0]


# speedup vs baseline: 19.1422x; 1.1569x over previous
import functools

import jax
import jax.numpy as jnp
from jax import lax
from jax.experimental import pallas as pl
from jax.experimental.pallas import tpu as pltpu
from jax.experimental.pallas import tpu_sc as plsc

THETA_F = 1.15
FILTER_F = 1e-08
NEG_BIG = -1e30

NC = 2
NS = 16
NW = NC * NS
LANES = 16
CHUNK = 128


def _tile_flatten(arr):
    B, C, H, W = arr.shape
    G = C * H // 8
    a4 = arr.reshape(B, G, 8, W)

    CT = W // 128

    def body(ref, oref):
        p, g = oref.shape[0], oref.shape[1]
        x = ref[...].reshape(p * g, 8, CT, 128)
        oref[...] = jnp.transpose(x, (0, 2, 1, 3)).reshape(p, g, CT * 8, 128)

    PB = 2
    return pl.pallas_call(
        body,
        grid=(B // PB,),
        in_specs=[pl.BlockSpec((PB, G, 8, W), lambda i: (i, 0, 0, 0))],
        out_specs=pl.BlockSpec((PB, G, CT * 8, 128),
                               lambda i: (i, 0, 0, 0)),
        out_shape=jax.ShapeDtypeStruct((B, G, CT * 8, 128),
                                       jnp.float32),
    )(a4)


def _sc_depth_targets(n_per_w):
    mesh = plsc.VectorSubcoreMesh(core_axis_name="c", subcore_axis_name="s")
    n_chunks = n_per_w // CHUNK
    n = NW * n_per_w

    @functools.partial(
        pl.kernel,
        mesh=mesh,
        out_type=[
            jax.ShapeDtypeStruct((n,), jnp.float32),
            jax.ShapeDtypeStruct((n,), jnp.int32),
            jax.ShapeDtypeStruct((n,), jnp.int32),
        ],
        scratch_types=[
            pltpu.VMEM((n_per_w,), jnp.int32),
            pltpu.VMEM((n_per_w,), jnp.int32),
            pltpu.VMEM((n_per_w,), jnp.float32),
            pltpu.VMEM((n_per_w,), jnp.float32),
            pltpu.VMEM((n_per_w,), jnp.float32),
            pltpu.SemaphoreType.DMA,
        ],
    )
    def k1(d_hbm, ia_hbm, ib_hbm, t_hbm, qa_hbm, qb_hbm,
           ia_v, ib_v, za_v, zb_v, t_v, sem):
        wid = lax.axis_index("s") * NC + lax.axis_index("c")
        base = wid * n_per_w
        pltpu.sync_copy(ia_hbm.at[pl.ds(base, n_per_w)], ia_v)
        pltpu.sync_copy(ib_hbm.at[pl.ds(base, n_per_w)], ib_v)
        handles = []
        for j in range(n_chunks):
            for iv in (ia_v, ib_v):
                for k in range(CHUNK // LANES):
                    sl = pl.ds(j * CHUNK + k * LANES, LANES)
                    f = iv[sl]
                    q = ((f & jnp.int32(-4096))
                         | (((f >> 7) & jnp.int32(3)) << 10)
                         | (((f >> 9) & jnp.int32(7)) << 7)
                         | (f & jnp.int32(127)))
                    iv[sl] = q
            s = pl.ds(j * CHUNK, CHUNK)
            handles.append(pltpu.async_copy(d_hbm.at[ia_v.at[s]], za_v.at[s], sem))
            handles.append(pltpu.async_copy(d_hbm.at[ib_v.at[s]], zb_v.at[s], sem))
        one = jnp.float32(1.0)
        neg_one = jnp.float32(-1.0)
        zero = jnp.float32(0.0)
        for j in range(n_chunks):
            handles[2 * j].wait()
            handles[2 * j + 1].wait()
            for k in range(CHUNK // LANES):
                s = pl.ds(j * CHUNK + k * LANES, LANES)
                za = za_v[s]
                zb = zb_v[s]
                keep = (za > FILTER_F) | (zb > FILTER_F)
                t = jnp.where(za / zb > THETA_F, neg_one,
                              jnp.where(zb / za > THETA_F, one, zero))
                t_v[s] = jnp.where(keep, t, zero)
        pltpu.sync_copy(t_v, t_hbm.at[pl.ds(base, n_per_w)])
        pltpu.sync_copy(ia_v, qa_hbm.at[pl.ds(base, n_per_w)])
        pltpu.sync_copy(ib_v, qb_hbm.at[pl.ds(base, n_per_w)])

    return k1


def _sc_pred_logits(n_per_w):
    mesh = plsc.VectorSubcoreMesh(core_axis_name="c", subcore_axis_name="s")
    n_chunks = n_per_w // CHUNK
    n = NW * n_per_w

    @functools.partial(
        pl.kernel,
        mesh=mesh,
        out_type=jax.ShapeDtypeStruct((n,), jnp.float32),
        scratch_types=[
            pltpu.VMEM((n_per_w,), jnp.int32),
            pltpu.VMEM((n_per_w,), jnp.int32),
            pltpu.VMEM((n_per_w,), jnp.float32),
            pltpu.VMEM((n_per_w,), jnp.float32),
            pltpu.VMEM((n_per_w,), jnp.float32),
            pltpu.VMEM((n_per_w,), jnp.float32),
            pltpu.SemaphoreType.DMA,
        ],
    )
    def k2(p_hbm, qa_hbm, qb_hbm, t_hbm, x_hbm,
           qa_v, qb_v, pa_v, pb_v, t_v, x_v, sem):
        wid = lax.axis_index("s") * NC + lax.axis_index("c")
        base = wid * n_per_w
        pltpu.sync_copy(qa_hbm.at[pl.ds(base, n_per_w)], qa_v)
        pltpu.sync_copy(qb_hbm.at[pl.ds(base, n_per_w)], qb_v)
        handles = []
        for j in range(n_chunks):
            s = pl.ds(j * CHUNK, CHUNK)
            handles.append(pltpu.async_copy(p_hbm.at[qa_v.at[s]], pa_v.at[s], sem))
            handles.append(pltpu.async_copy(p_hbm.at[qb_v.at[s]], pb_v.at[s], sem))
        pltpu.sync_copy(t_hbm.at[pl.ds(base, n_per_w)], t_v)
        zero = jnp.float32(0.0)
        for j in range(n_chunks):
            handles[2 * j].wait()
            handles[2 * j + 1].wait()
            for k in range(CHUNK // LANES):
                s = pl.ds(j * CHUNK + k * LANES, LANES)
                t = t_v[s]
                x = jnp.where(t != zero, -t * (pa_v[s] - pb_v[s]),
                              jnp.float32(NEG_BIG))
                x_v[s] = x
        pltpu.sync_copy(x_v, x_hbm.at[pl.ds(base, n_per_w)])

    return k2


def _softplus_sum(x_ref, o_ref):
    x = x_ref[...]
    o_ref[...] = jnp.sum(jnp.log1p(jnp.exp(x))).reshape(1, 1)


def kernel(depth, pred, idx_A, idx_B):
    n = idx_A.shape[0]
    n_per_w = n // NW
    d_flat = _tile_flatten(depth).reshape(-1)
    t, qa, qb = _sc_depth_targets(n_per_w)(d_flat, idx_A, idx_B)
    p_flat = _tile_flatten(pred).reshape(-1)
    x = _sc_pred_logits(n_per_w)(p_flat, qa, qb, t)
    loss = pl.pallas_call(
        _softplus_sum,
        out_shape=jax.ShapeDtypeStruct((1, 1), jnp.float32),
    )(x.reshape(n // CHUNK, CHUNK))
    return loss[0, 0]
